# Initial kernel scaffold; baseline (speedup 1.0000x reference)
import jax
import jax.numpy as jnp
from jax.experimental import pallas as pl


def kernel(collocation, W):
    raise NotImplementedError("write your pallas kernel here")


# speedup vs baseline: 13.5548x; 13.5548x over previous
import jax
import jax.numpy as jnp
from jax import lax
from jax.experimental import pallas as pl
from jax.experimental.pallas import tpu as pltpu
from jax.experimental.pallas import tpu_sc as plsc

_DEPTH = 8
_NUM_LEAVES = 256
_SIZE = 511
_GCOLS = 512
_NDIMS = 2048
_BATCH = 4096
_NW = 32
_BPW = _BATCH // _NW
_LANES = 16
_NPATH = _DEPTH + 1


def _g_matmul_body(a_ref, b_ref, g_ref):
    g_ref[...] = lax.dot_general(
        a_ref[...], b_ref[...],
        dimension_numbers=(((1,), (0,)), ((), ())),
        preferred_element_type=jnp.float32,
        precision=lax.Precision.HIGHEST,
    )


def _sc_probs_body(c0_hbm, c1_hbm, g_hbm, out_hbm,
                   c0_v, c1_v, idx_v, vals_v, out_v, sem):
    wid = lax.axis_index("s") * 2 + lax.axis_index("c")
    base = wid * _BPW
    pltpu.sync_copy(c0_hbm.at[pl.ds(base, _BPW)], c0_v)
    pltpu.sync_copy(c1_hbm.at[pl.ds(base, _BPW)], c1_v)
    for g in range(_BPW // _LANES):
        c0g = c0_v[pl.ds(g * _LANES, _LANES)]
        t = c1_v[pl.ds(g * _LANES, _LANES)] + 256
        for d in range(_NPATH):
            node = jnp.right_shift(t, _DEPTH - d) - 1
            idx_v[d, pl.ds(g * _LANES, _LANES)] = c0g * _GCOLS + node
    copies = [
        pltpu.async_copy(g_hbm.at[idx_v.at[d]], vals_v.at[d], sem)
        for d in range(_NPATH)
    ]
    for c in copies:
        c.wait()
    for g in range(_BPW // _LANES):
        acc = jnp.ones((_LANES,), jnp.float32)
        for d in range(_NPATH):
            v = vals_v[d, pl.ds(g * _LANES, _LANES)]
            acc = acc * (1.0 / (1.0 + jnp.exp(-v)))
        out_v[pl.ds(g * _LANES, _LANES)] = acc
    pltpu.sync_copy(out_v, out_hbm.at[pl.ds(base, _BPW)])


def kernel(collocation, W):
    coll = collocation.astype(jnp.int32)
    c0 = coll[:, 0]
    c1 = coll[:, 1]
    w_leaf = lax.slice(W, (_SIZE - _NUM_LEAVES, 0), (_SIZE, _NDIMS))
    w_pad_t = jnp.pad(W, ((0, _GCOLS - _SIZE), (0, 0))).T

    g_table = pl.pallas_call(
        _g_matmul_body,
        out_shape=jax.ShapeDtypeStruct((_NUM_LEAVES, _GCOLS), jnp.float32),
    )(w_leaf, w_pad_t)
    g_flat = g_table.reshape(_NUM_LEAVES * _GCOLS)

    mesh = plsc.VectorSubcoreMesh(core_axis_name="c", subcore_axis_name="s")
    sc_probs = pl.kernel(
        _sc_probs_body,
        out_type=jax.ShapeDtypeStruct((_BATCH,), jnp.float32),
        mesh=mesh,
        scratch_types=[
            pltpu.VMEM((_BPW,), jnp.int32),
            pltpu.VMEM((_BPW,), jnp.int32),
            pltpu.VMEM((_NPATH, _BPW), jnp.int32),
            pltpu.VMEM((_NPATH, _BPW), jnp.float32),
            pltpu.VMEM((_BPW,), jnp.float32),
            pltpu.SemaphoreType.DMA,
        ],
        compiler_params=pltpu.CompilerParams(use_tc_tiling_on_sc=False),
    )
    return sc_probs(c0, c1, g_flat)


# speedup vs baseline: 18.3885x; 1.3566x over previous
import jax
import jax.numpy as jnp
from jax import lax
from jax.experimental import pallas as pl
from jax.experimental.pallas import tpu as pltpu
from jax.experimental.pallas import tpu_sc as plsc

_DEPTH = 8
_NUM_LEAVES = 256
_SIZE = 511
_GCOLS = 512
_NDIMS = 2048
_BATCH = 4096
_NW = 32
_BPW = _BATCH // _NW
_LANES = 16
_NPATH = _DEPTH + 1


def _g_matmul_body(w_ref, g_ref):
    leaf = w_ref[pl.ds(_SIZE - _NUM_LEAVES, _NUM_LEAVES), :]
    g = lax.dot_general(
        leaf, w_ref[...],
        dimension_numbers=(((1,), (1,)), ((), ())),
        preferred_element_type=jnp.float32,
        precision=lax.Precision.HIGHEST,
    )
    g_ref[...] = g.reshape(_NUM_LEAVES * _GCOLS)


def _sc_probs_body(c0_hbm, c1_hbm, g_hbm, out_hbm,
                   c0_v, c1_v, idx_v, vals_v, out_v, sem):
    wid = lax.axis_index("s") * 2 + lax.axis_index("c")
    base = wid * _BPW
    pltpu.sync_copy(c0_hbm.at[pl.ds(base, _BPW)], c0_v)
    pltpu.sync_copy(c1_hbm.at[pl.ds(base, _BPW)], c1_v)
    for g in range(_BPW // _LANES):
        c0g = c0_v[pl.ds(g * _LANES, _LANES)]
        t = c1_v[pl.ds(g * _LANES, _LANES)] + 256
        for d in range(_NPATH):
            node = jnp.right_shift(t, _DEPTH - d) - 1
            idx_v[d, pl.ds(g * _LANES, _LANES)] = c0g * _GCOLS + node
    copies = [
        pltpu.async_copy(g_hbm.at[idx_v.at[d]], vals_v.at[d], sem)
        for d in range(_NPATH)
    ]
    for c in copies:
        c.wait()
    for g in range(_BPW // _LANES):
        acc = jnp.ones((_LANES,), jnp.float32)
        for d in range(_NPATH):
            v = vals_v[d, pl.ds(g * _LANES, _LANES)]
            acc = acc * (1.0 / (1.0 + jnp.exp(-v)))
        out_v[pl.ds(g * _LANES, _LANES)] = acc
    pltpu.sync_copy(out_v, out_hbm.at[pl.ds(base, _BPW)])


def kernel(collocation, W):
    coll = collocation.astype(jnp.int32)
    c0 = coll[:, 0]
    c1 = coll[:, 1]
    g_flat = pl.pallas_call(
        _g_matmul_body,
        grid=(1,),
        in_specs=[pl.BlockSpec((_GCOLS, _NDIMS), lambda i: (0, 0))],
        out_specs=pl.BlockSpec((_NUM_LEAVES * _GCOLS,), lambda i: (0,)),
        out_shape=jax.ShapeDtypeStruct((_NUM_LEAVES * _GCOLS,), jnp.float32),
    )(W)

    mesh = plsc.VectorSubcoreMesh(core_axis_name="c", subcore_axis_name="s")
    sc_probs = pl.kernel(
        _sc_probs_body,
        out_type=jax.ShapeDtypeStruct((_BATCH,), jnp.float32),
        mesh=mesh,
        scratch_types=[
            pltpu.VMEM((_BPW,), jnp.int32),
            pltpu.VMEM((_BPW,), jnp.int32),
            pltpu.VMEM((_NPATH, _BPW), jnp.int32),
            pltpu.VMEM((_NPATH, _BPW), jnp.float32),
            pltpu.VMEM((_BPW,), jnp.float32),
            pltpu.SemaphoreType.DMA,
        ],
        compiler_params=pltpu.CompilerParams(use_tc_tiling_on_sc=False),
    )
    return sc_probs(c0, c1, g_flat)


# speedup vs baseline: 20.5638x; 1.1183x over previous
import jax
import jax.numpy as jnp
from jax import lax
from jax.experimental import pallas as pl
from jax.experimental.pallas import tpu as pltpu
from jax.experimental.pallas import tpu_sc as plsc

_DEPTH = 8
_NUM_LEAVES = 256
_SIZE = 511
_GCOLS = 512
_NDIMS = 2048
_BATCH = 4096
_NW = 32
_BPW = _BATCH // _NW
_LANES = 16
_NPATH = _DEPTH + 1


def _g_matmul_body(w_ref, g_ref):
    leaf = w_ref[pl.ds(_SIZE - _NUM_LEAVES, _NUM_LEAVES), :]
    g = lax.dot_general(
        leaf, w_ref[...],
        dimension_numbers=(((1,), (1,)), ((), ())),
        preferred_element_type=jnp.float32,
        precision=lax.Precision.DEFAULT,
    )
    g_ref[...] = g.reshape(_NUM_LEAVES * _GCOLS)


def _sc_probs_body(c0_hbm, c1_hbm, g_hbm, out_hbm,
                   c0_v, c1_v, idx_v, vals_v, out_v, sem):
    wid = lax.axis_index("s") * 2 + lax.axis_index("c")
    base = wid * _BPW
    pltpu.sync_copy(c0_hbm.at[pl.ds(base, _BPW)], c0_v)
    pltpu.sync_copy(c1_hbm.at[pl.ds(base, _BPW)], c1_v)
    for g in range(_BPW // _LANES):
        c0g = c0_v[pl.ds(g * _LANES, _LANES)]
        t = c1_v[pl.ds(g * _LANES, _LANES)] + 256
        for d in range(_NPATH):
            node = jnp.right_shift(t, _DEPTH - d) - 1
            idx_v[d, pl.ds(g * _LANES, _LANES)] = c0g * _GCOLS + node
    copies = [
        pltpu.async_copy(g_hbm.at[idx_v.at[d]], vals_v.at[d], sem)
        for d in range(_NPATH)
    ]
    for c in copies:
        c.wait()
    for g in range(_BPW // _LANES):
        acc = jnp.ones((_LANES,), jnp.float32)
        for d in range(_NPATH):
            v = vals_v[d, pl.ds(g * _LANES, _LANES)]
            acc = acc * (1.0 / (1.0 + jnp.exp(-v)))
        out_v[pl.ds(g * _LANES, _LANES)] = acc
    pltpu.sync_copy(out_v, out_hbm.at[pl.ds(base, _BPW)])


def kernel(collocation, W):
    coll = collocation.astype(jnp.int32)
    c0 = coll[:, 0]
    c1 = coll[:, 1]
    g_flat = pl.pallas_call(
        _g_matmul_body,
        grid=(1,),
        in_specs=[pl.BlockSpec((_GCOLS, _NDIMS), lambda i: (0, 0))],
        out_specs=pl.BlockSpec((_NUM_LEAVES * _GCOLS,), lambda i: (0,)),
        out_shape=jax.ShapeDtypeStruct((_NUM_LEAVES * _GCOLS,), jnp.float32),
    )(W)

    mesh = plsc.VectorSubcoreMesh(core_axis_name="c", subcore_axis_name="s")
    sc_probs = pl.kernel(
        _sc_probs_body,
        out_type=jax.ShapeDtypeStruct((_BATCH,), jnp.float32),
        mesh=mesh,
        scratch_types=[
            pltpu.VMEM((_BPW,), jnp.int32),
            pltpu.VMEM((_BPW,), jnp.int32),
            pltpu.VMEM((_NPATH, _BPW), jnp.int32),
            pltpu.VMEM((_NPATH, _BPW), jnp.float32),
            pltpu.VMEM((_BPW,), jnp.float32),
            pltpu.SemaphoreType.DMA,
        ],
        compiler_params=pltpu.CompilerParams(use_tc_tiling_on_sc=False),
    )
    return sc_probs(c0, c1, g_flat)
